# baseline (speedup 1.0000x reference)
import functools

import jax
import jax.numpy as jnp
from jax.experimental import pallas as pl
from jax.experimental.pallas import tpu as pltpu

K = 8192
DIM = 32
BT = 256


def _norms_block(cb0_ref, cb1_ref, cb2_ref, w0_ref, w1_ref, w2_ref):
    for cb_ref, w_ref in ((cb0_ref, w0_ref), (cb1_ref, w1_ref),
                          (cb2_ref, w2_ref)):
        W = cb_ref[...]
        w_ref[...] = jnp.sum(W * W, axis=1, keepdims=True)


def _rvq_block(z_ref, t0_ref, t1_ref, t2_ref, w0_ref, w1_ref, w2_ref,
               zhat_ref, i0_ref, i1_ref, i2_ref):
    z = z_ref[...]
    residual = z
    z_hat = jnp.zeros_like(z)
    for wt_ref, ww_ref, i_ref in ((t0_ref, w0_ref, i0_ref),
                                  (t1_ref, w1_ref, i1_ref),
                                  (t2_ref, w2_ref, i2_ref)):
        Wt2 = wt_ref[...]
        ww = ww_ref[...]
        rr = jnp.sum(residual * residual, axis=1, keepdims=True)
        mm2 = jax.lax.dot_general(
            residual, Wt2, (((1,), (0,)), ((), ())),
            preferred_element_type=jnp.float32)
        d2 = rr - mm2 + ww
        m = jnp.min(d2, axis=1, keepdims=True)
        iota = jax.lax.broadcasted_iota(jnp.int32, d2.shape, 1)
        idx = jnp.min(jnp.where(d2 == m, iota, K), axis=1, keepdims=True)
        onehot = (iota == idx).astype(jnp.float32)
        z_q = 0.5 * jax.lax.dot_general(
            onehot, Wt2, (((1,), (1,)), ((), ())),
            preferred_element_type=jnp.float32)
        quant_st = residual + (z_q - residual)
        z_hat = z_hat + quant_st
        residual = residual - quant_st
        i_ref[...] = idx
    zhat_ref[...] = z_hat


def kernel(z, codebook0, codebook1, codebook2):
    B = z.shape[0]
    cb_spec = pl.BlockSpec((K, DIM), lambda: (0, 0))
    ww_col = pl.pallas_call(
        _norms_block,
        in_specs=[cb_spec, cb_spec, cb_spec],
        out_specs=[pl.BlockSpec((K, 1), lambda: (0, 0))] * 3,
        out_shape=[jax.ShapeDtypeStruct((K, 1), jnp.float32)] * 3,
    )(codebook0, codebook1, codebook2)
    wws = [w.reshape(1, K) for w in ww_col]
    wts = [2.0 * cb.T for cb in (codebook0, codebook1, codebook2)]

    grid = (B // BT,)
    wt_spec = pl.BlockSpec((DIM, K), lambda i: (0, 0))
    ww_spec = pl.BlockSpec((1, K), lambda i: (0, 0))
    z_hat, i0, i1, i2 = pl.pallas_call(
        _rvq_block,
        grid=grid,
        in_specs=[pl.BlockSpec((BT, DIM), lambda i: (i, 0)),
                  wt_spec, wt_spec, wt_spec,
                  ww_spec, ww_spec, ww_spec],
        out_specs=[
            pl.BlockSpec((BT, DIM), lambda i: (i, 0)),
            pl.BlockSpec((BT, 1), lambda i: (i, 0)),
            pl.BlockSpec((BT, 1), lambda i: (i, 0)),
            pl.BlockSpec((BT, 1), lambda i: (i, 0)),
        ],
        out_shape=[
            jax.ShapeDtypeStruct((B, DIM), jnp.float32),
            jax.ShapeDtypeStruct((B, 1), jnp.int32),
            jax.ShapeDtypeStruct((B, 1), jnp.int32),
            jax.ShapeDtypeStruct((B, 1), jnp.int32),
        ],
        compiler_params=pltpu.CompilerParams(
            dimension_semantics=("parallel",)),
    )(z, *wts, *wws)
    indices = jnp.concatenate([i0, i1, i2], axis=1)
    return z_hat, indices


# speedup vs baseline: 1.9153x; 1.9153x over previous
import functools

import jax
import jax.numpy as jnp
from jax import lax
from jax.experimental import pallas as pl
from jax.experimental.pallas import tpu as pltpu
from jax.experimental.pallas import tpu_sc as plsc

K = 8192
DIM = 32
BT = 256
BTC = 2048


def _norms_block(cb0_ref, cb1_ref, cb2_ref, w0_ref, w1_ref, w2_ref):
    for cb_ref, w_ref in ((cb0_ref, w0_ref), (cb1_ref, w1_ref),
                          (cb2_ref, w2_ref)):
        W = cb_ref[...]
        w_ref[...] = jnp.sum(W * W, axis=1, keepdims=True)


def _residual_chain(z, zqs):
    r = z
    z_hat = jnp.zeros_like(z)
    for zq in zqs:
        quant_st = r + (zq - r)
        z_hat = z_hat + quant_st
        r = r - quant_st
    return r, z_hat


def _level_block(nprev, *refs):
    z_ref = refs[0]
    zq_refs = refs[1:1 + nprev]
    wt_ref, ww_ref, i_ref = refs[1 + nprev:]
    r, _ = _residual_chain(z_ref[...],
                           [q[...][:, :DIM] for q in zq_refs])
    Wt2 = wt_ref[...]
    ww = ww_ref[...]
    rr = jnp.sum(r * r, axis=1, keepdims=True)
    mm2 = jax.lax.dot_general(
        r, Wt2, (((1,), (0,)), ((), ())),
        preferred_element_type=jnp.float32)
    d2 = rr - mm2 + ww
    m = jnp.min(d2, axis=1, keepdims=True)
    iota = jax.lax.broadcasted_iota(jnp.int32, d2.shape, 1)
    i_ref[...] = jnp.min(jnp.where(d2 == m, iota, K), axis=1, keepdims=True)


def _combine_block(z_ref, q0_ref, q1_ref, q2_ref, zhat_ref):
    _, z_hat = _residual_chain(
        z_ref[...],
        [q[...][:, :DIM] for q in (q0_ref, q1_ref, q2_ref)])
    zhat_ref[...] = z_hat


PADW = 128


def _make_sc_gather(B):
    info = plsc.get_sparse_core_info()
    NC, NS = info.num_cores, info.num_subcores
    NW = NC * NS
    b_per_w = B // NW
    ch = 512
    mesh = plsc.VectorSubcoreMesh(core_axis_name="c", subcore_axis_name="s")

    @functools.partial(
        pl.kernel, mesh=mesh,
        out_type=jax.ShapeDtypeStruct((B, PADW), jnp.float32),
        scratch_types=[
            pltpu.VMEM((b_per_w,), jnp.int32),
            pltpu.VMEM((ch, PADW), jnp.float32),
            pltpu.SemaphoreType.DMA,
        ],
    )
    def gather_k(table_hbm, idx_hbm, out_hbm, idx_v, rows_v, sem):
        wid = lax.axis_index("s") * NC + lax.axis_index("c")
        base = wid * b_per_w
        pltpu.sync_copy(idx_hbm.at[pl.ds(base, b_per_w)], idx_v)
        for c in range(b_per_w // ch):
            pltpu.async_copy(
                table_hbm.at[idx_v.at[pl.ds(c * ch, ch)]], rows_v, sem
            ).wait()
            pltpu.sync_copy(rows_v, out_hbm.at[pl.ds(base + c * ch, ch)])

    return gather_k


def kernel(z, codebook0, codebook1, codebook2):
    B = z.shape[0]
    cbs = (codebook0, codebook1, codebook2)
    cb_spec = pl.BlockSpec((K, DIM), lambda: (0, 0))
    ww_col = pl.pallas_call(
        _norms_block,
        in_specs=[cb_spec, cb_spec, cb_spec],
        out_specs=[pl.BlockSpec((K, 1), lambda: (0, 0))] * 3,
        out_shape=[jax.ShapeDtypeStruct((K, 1), jnp.float32)] * 3,
    )(*cbs)
    wws = [w.reshape(1, K) for w in ww_col]
    wts = [2.0 * cb.T for cb in cbs]

    grid = (B // BT,)
    tok_spec = pl.BlockSpec((BT, DIM), lambda i: (i, 0))
    zq_spec = pl.BlockSpec((BT, PADW), lambda i: (i, 0))
    wt_spec = pl.BlockSpec((DIM, K), lambda i: (0, 0))
    ww_spec = pl.BlockSpec((1, K), lambda i: (0, 0))
    idx_shape = jax.ShapeDtypeStruct((B, 1), jnp.int32)
    idx_spec = pl.BlockSpec((BT, 1), lambda i: (i, 0))

    def level(nprev, zqs, wt, ww):
        return pl.pallas_call(
            functools.partial(_level_block, nprev),
            grid=grid,
            in_specs=[tok_spec] + [zq_spec] * nprev + [wt_spec, ww_spec],
            out_specs=idx_spec,
            out_shape=idx_shape,
            compiler_params=pltpu.CompilerParams(
                dimension_semantics=("arbitrary",)),
        )(z, *zqs, wt, ww)

    sc_gather = _make_sc_gather(B)
    padded = [jnp.pad(cb, ((0, 0), (0, PADW - DIM))) for cb in cbs]

    i0 = level(0, [], wts[0], wws[0])
    zq0 = sc_gather(padded[0], i0.reshape(B))
    i1 = level(1, [zq0], wts[1], wws[1])
    zq1 = sc_gather(padded[1], i1.reshape(B))
    i2 = level(2, [zq0, zq1], wts[2], wws[2])
    zq2 = sc_gather(padded[2], i2.reshape(B))

    ctok = pl.BlockSpec((BTC, DIM), lambda i: (i, 0))
    cq = pl.BlockSpec((BTC, PADW), lambda i: (i, 0))
    z_hat = pl.pallas_call(
        _combine_block,
        grid=(B // BTC,),
        in_specs=[ctok, cq, cq, cq],
        out_specs=ctok,
        out_shape=jax.ShapeDtypeStruct((B, DIM), jnp.float32),
    )(z, zq0, zq1, zq2)

    indices = jnp.concatenate([i0, i1, i2], axis=1)
    return z_hat, indices


# speedup vs baseline: 2.0153x; 1.0523x over previous
import functools

import jax
import jax.numpy as jnp
from jax import lax
from jax.experimental import pallas as pl
from jax.experimental.pallas import tpu as pltpu
from jax.experimental.pallas import tpu_sc as plsc

K = 8192
DIM = 32
BT = 512
BTC = 2048


def _norms_block(cb0_ref, cb1_ref, cb2_ref, w0_ref, w1_ref, w2_ref):
    for cb_ref, w_ref in ((cb0_ref, w0_ref), (cb1_ref, w1_ref),
                          (cb2_ref, w2_ref)):
        W = cb_ref[...]
        w_ref[...] = jnp.sum(W * W, axis=1, keepdims=True)


def _residual_chain(z, zqs):
    r = z
    z_hat = jnp.zeros_like(z)
    for zq in zqs:
        quant_st = r + (zq - r)
        z_hat = z_hat + quant_st
        r = r - quant_st
    return r, z_hat


def _level_block(nprev, *refs):
    z_ref = refs[0]
    zq_refs = refs[1:1 + nprev]
    wt_ref, ww_ref, i_ref = refs[1 + nprev:]
    r, _ = _residual_chain(z_ref[...],
                           [q[...][:, :DIM] for q in zq_refs])
    Wt2 = wt_ref[...]
    ww = ww_ref[...]
    rr = jnp.sum(r * r, axis=1, keepdims=True)
    mm2 = jax.lax.dot_general(
        r, Wt2, (((1,), (0,)), ((), ())),
        preferred_element_type=jnp.float32)
    d2 = rr - mm2 + ww
    m = jnp.min(d2, axis=1, keepdims=True)
    iota = jax.lax.broadcasted_iota(jnp.int32, d2.shape, 1)
    i_ref[...] = jnp.min(jnp.where(d2 == m, iota, K), axis=1, keepdims=True)


def _combine_block(z_ref, q0_ref, q1_ref, q2_ref, zhat_ref):
    _, z_hat = _residual_chain(
        z_ref[...],
        [q[...][:, :DIM] for q in (q0_ref, q1_ref, q2_ref)])
    zhat_ref[...] = z_hat


PADW = 128


def _make_sc_gather(B):
    info = plsc.get_sparse_core_info()
    NC, NS = info.num_cores, info.num_subcores
    NW = NC * NS
    b_per_w = B // NW
    ch = 512
    mesh = plsc.VectorSubcoreMesh(core_axis_name="c", subcore_axis_name="s")

    @functools.partial(
        pl.kernel, mesh=mesh,
        out_type=jax.ShapeDtypeStruct((B, PADW), jnp.float32),
        scratch_types=[
            pltpu.VMEM((b_per_w,), jnp.int32),
            pltpu.VMEM((ch, PADW), jnp.float32),
            pltpu.SemaphoreType.DMA,
        ],
    )
    def gather_k(table_hbm, idx_hbm, out_hbm, idx_v, rows_v, sem):
        wid = lax.axis_index("s") * NC + lax.axis_index("c")
        base = wid * b_per_w
        pltpu.sync_copy(idx_hbm.at[pl.ds(base, b_per_w)], idx_v)
        for c in range(b_per_w // ch):
            pltpu.async_copy(
                table_hbm.at[idx_v.at[pl.ds(c * ch, ch)]], rows_v, sem
            ).wait()
            pltpu.sync_copy(rows_v, out_hbm.at[pl.ds(base + c * ch, ch)])

    return gather_k


def kernel(z, codebook0, codebook1, codebook2):
    B = z.shape[0]
    cbs = (codebook0, codebook1, codebook2)
    cb_spec = pl.BlockSpec((K, DIM), lambda: (0, 0))
    ww_col = pl.pallas_call(
        _norms_block,
        in_specs=[cb_spec, cb_spec, cb_spec],
        out_specs=[pl.BlockSpec((K, 1), lambda: (0, 0))] * 3,
        out_shape=[jax.ShapeDtypeStruct((K, 1), jnp.float32)] * 3,
    )(*cbs)
    wws = [w.reshape(1, K) for w in ww_col]
    wts = [2.0 * cb.T for cb in cbs]

    H = B // 2
    tok_spec = pl.BlockSpec((BT, DIM), lambda i: (i, 0))
    zq_spec = pl.BlockSpec((BT, PADW), lambda i: (i, 0))
    wt_spec = pl.BlockSpec((DIM, K), lambda i: (0, 0))
    ww_spec = pl.BlockSpec((1, K), lambda i: (0, 0))
    idx_spec = pl.BlockSpec((BT, 1), lambda i: (i, 0))

    def level(nprev, zh, zqs, wt, ww):
        return pl.pallas_call(
            functools.partial(_level_block, nprev),
            grid=(H // BT,),
            in_specs=[tok_spec] + [zq_spec] * nprev + [wt_spec, ww_spec],
            out_specs=idx_spec,
            out_shape=jax.ShapeDtypeStruct((H, 1), jnp.int32),
            compiler_params=pltpu.CompilerParams(
                dimension_semantics=("arbitrary",)),
        )(zh, *zqs, wt, ww)

    sc_gather = _make_sc_gather(H)
    padded = [jnp.pad(cb, ((0, 0), (0, PADW - DIM))) for cb in cbs]
    zhalf = [z[:H], z[H:]]

    idx = [[None, None, None] for _ in range(2)]
    zq = [[None, None, None] for _ in range(2)]
    for lvl in range(3):
        for h in range(2):
            idx[h][lvl] = level(lvl, zhalf[h], zq[h][:lvl],
                                wts[lvl], wws[lvl])
            zq[h][lvl] = sc_gather(padded[lvl], idx[h][lvl].reshape(H))

    ctok = pl.BlockSpec((BTC, DIM), lambda i: (i, 0))
    cq = pl.BlockSpec((BTC, PADW), lambda i: (i, 0))
    zh_half = [
        pl.pallas_call(
            _combine_block,
            grid=(H // BTC,),
            in_specs=[ctok, cq, cq, cq],
            out_specs=ctok,
            out_shape=jax.ShapeDtypeStruct((H, DIM), jnp.float32),
        )(zhalf[h], *zq[h])
        for h in range(2)
    ]
    z_hat = jnp.concatenate(zh_half, axis=0)
    indices = jnp.concatenate(
        [jnp.concatenate(idx[h], axis=1) for h in range(2)], axis=0)
    return z_hat, indices


# speedup vs baseline: 2.2505x; 1.1167x over previous
import functools

import jax
import jax.numpy as jnp
from jax import lax
from jax.experimental import pallas as pl
from jax.experimental.pallas import tpu as pltpu
from jax.experimental.pallas import tpu_sc as plsc

K = 8192
DIM = 32
BT = 512
BTC = 2048


def _norms_block(cb0_ref, cb1_ref, cb2_ref, w0_ref, w1_ref, w2_ref):
    for cb_ref, w_ref in ((cb0_ref, w0_ref), (cb1_ref, w1_ref),
                          (cb2_ref, w2_ref)):
        W = cb_ref[...]
        w_ref[...] = jnp.sum(W * W, axis=1, keepdims=True)


def _residual_chain(z, zqs):
    r = z
    z_hat = jnp.zeros_like(z)
    for zq in zqs:
        quant_st = r + (zq - r)
        z_hat = z_hat + quant_st
        r = r - quant_st
    return r, z_hat


def _level_block(nprev, *refs):
    z_ref = refs[0]
    zq_refs = refs[1:1 + nprev]
    wt_ref, ww_ref, iota_ref, i_ref = refs[1 + nprev:]
    r, _ = _residual_chain(z_ref[...],
                           [q[...][:, :DIM] for q in zq_refs])
    Wt2 = wt_ref[...]
    ww = ww_ref[...]
    iotaf = iota_ref[...]
    rr = jnp.sum(r * r, axis=1, keepdims=True)
    mm2 = jax.lax.dot_general(
        r, Wt2, (((1,), (0,)), ((), ())),
        preferred_element_type=jnp.float32)
    d2 = rr - mm2 + ww
    m = jnp.min(d2, axis=1, keepdims=True)
    idxf = jnp.min(jnp.where(d2 == m, iotaf, jnp.float32(K)),
                   axis=1, keepdims=True)
    i_ref[...] = idxf.astype(jnp.int32)


def _combine_block(z_ref, q0_ref, q1_ref, q2_ref, zhat_ref):
    _, z_hat = _residual_chain(
        z_ref[...],
        [q[...][:, :DIM] for q in (q0_ref, q1_ref, q2_ref)])
    zhat_ref[...] = z_hat


PADW = 128


def _make_sc_gather(B):
    info = plsc.get_sparse_core_info()
    NC, NS = info.num_cores, info.num_subcores
    NW = NC * NS
    b_per_w = B // NW
    ch = 512
    mesh = plsc.VectorSubcoreMesh(core_axis_name="c", subcore_axis_name="s")

    @functools.partial(
        pl.kernel, mesh=mesh,
        out_type=jax.ShapeDtypeStruct((B, PADW), jnp.float32),
        scratch_types=[
            pltpu.VMEM((b_per_w,), jnp.int32),
            pltpu.VMEM((ch, PADW), jnp.float32),
            pltpu.SemaphoreType.DMA,
        ],
    )
    def gather_k(table_hbm, idx_hbm, out_hbm, idx_v, rows_v, sem):
        wid = lax.axis_index("s") * NC + lax.axis_index("c")
        base = wid * b_per_w
        pltpu.sync_copy(idx_hbm.at[pl.ds(base, b_per_w)], idx_v)
        for c in range(b_per_w // ch):
            pltpu.async_copy(
                table_hbm.at[idx_v.at[pl.ds(c * ch, ch)]], rows_v, sem
            ).wait()
            pltpu.sync_copy(rows_v, out_hbm.at[pl.ds(base + c * ch, ch)])

    return gather_k


def kernel(z, codebook0, codebook1, codebook2):
    B = z.shape[0]
    cbs = (codebook0, codebook1, codebook2)
    cb_spec = pl.BlockSpec((K, DIM), lambda: (0, 0))
    ww_col = pl.pallas_call(
        _norms_block,
        in_specs=[cb_spec, cb_spec, cb_spec],
        out_specs=[pl.BlockSpec((K, 1), lambda: (0, 0))] * 3,
        out_shape=[jax.ShapeDtypeStruct((K, 1), jnp.float32)] * 3,
    )(*cbs)
    wws = [w.reshape(1, K) for w in ww_col]
    wts = [2.0 * cb.T for cb in cbs]

    H = B // 2
    iotaf = jnp.arange(K, dtype=jnp.float32).reshape(1, K)
    tok_spec = pl.BlockSpec((BT, DIM), lambda i: (i, 0))
    wt_spec = pl.BlockSpec((DIM, K), lambda i: (0, 0))
    ww_spec = pl.BlockSpec((1, K), lambda i: (0, 0))
    idx_spec = pl.BlockSpec((BT, 1), lambda i: (i, 0))

    zq_spec = pl.BlockSpec((BT, PADW), lambda i: (i, 0))

    def level(nprev, zh, zqs, wt, ww):
        return pl.pallas_call(
            functools.partial(_level_block, nprev),
            grid=(H // BT,),
            in_specs=[tok_spec] + [zq_spec] * nprev
                     + [wt_spec, ww_spec, ww_spec],
            out_specs=idx_spec,
            out_shape=jax.ShapeDtypeStruct((H, 1), jnp.int32),
            compiler_params=pltpu.CompilerParams(
                dimension_semantics=("arbitrary",)),
        )(zh, *zqs, wt, ww, iotaf)

    sc_gather = _make_sc_gather(H)
    padded = [jnp.pad(cb, ((0, 0), (0, PADW - DIM))) for cb in cbs]
    zhalf = [z[:H], z[H:]]

    idx = [[None, None, None] for _ in range(2)]
    zq = [[None, None, None] for _ in range(2)]
    for lvl in range(3):
        for h in range(2):
            idx[h][lvl] = level(lvl, zhalf[h], zq[h][:lvl],
                                wts[lvl], wws[lvl])
            zq[h][lvl] = sc_gather(padded[lvl], idx[h][lvl].reshape(H))

    ctok = pl.BlockSpec((BTC, DIM), lambda i: (i, 0))
    cq = pl.BlockSpec((BTC, PADW), lambda i: (i, 0))
    zh_half = [
        pl.pallas_call(
            _combine_block,
            grid=(H // BTC,),
            in_specs=[ctok, cq, cq, cq],
            out_specs=ctok,
            out_shape=jax.ShapeDtypeStruct((H, DIM), jnp.float32),
        )(zhalf[h], *zq[h])
        for h in range(2)
    ]
    z_hat = jnp.concatenate(zh_half, axis=0)
    indices = jnp.concatenate(
        [jnp.concatenate(idx[h], axis=1) for h in range(2)], axis=0)
    return z_hat, indices


# speedup vs baseline: 2.2521x; 1.0007x over previous
import functools

import jax
import jax.numpy as jnp
from jax import lax
from jax.experimental import pallas as pl
from jax.experimental.pallas import tpu as pltpu
from jax.experimental.pallas import tpu_sc as plsc

K = 8192
DIM = 32
BT = 512
BTC = 2048


def _norms_block(cb0_ref, cb1_ref, cb2_ref, w0_ref, w1_ref, w2_ref):
    for cb_ref, w_ref in ((cb0_ref, w0_ref), (cb1_ref, w1_ref),
                          (cb2_ref, w2_ref)):
        W = cb_ref[...]
        w_ref[...] = jnp.sum(W * W, axis=1, keepdims=True)


def _residual_chain(z, zqs):
    r = z
    z_hat = jnp.zeros_like(z)
    for zq in zqs:
        quant_st = r + (zq - r)
        z_hat = z_hat + quant_st
        r = r - quant_st
    return r, z_hat


def _level_block(nprev, *refs):
    z_ref = refs[0]
    zq_refs = refs[1:1 + nprev]
    wt_ref, ww_ref, iota_ref, i_ref = refs[1 + nprev:]
    r, _ = _residual_chain(z_ref[...],
                           [q[...][:, :DIM] for q in zq_refs])
    Wt2 = wt_ref[...]
    ww = ww_ref[...]
    iotaf = iota_ref[...]
    rr = jnp.sum(r * r, axis=1, keepdims=True)
    mm2 = jax.lax.dot_general(
        r, Wt2, (((1,), (0,)), ((), ())),
        preferred_element_type=jnp.float32)
    d2 = rr - mm2 + ww
    m = jnp.min(d2, axis=1, keepdims=True)
    idxf = jnp.min(jnp.where(d2 == m, iotaf, jnp.float32(K)),
                   axis=1, keepdims=True)
    i_ref[...] = idxf.astype(jnp.int32)


def _combine_block(z_ref, q0_ref, q1_ref, q2_ref, zhat_ref):
    _, z_hat = _residual_chain(
        z_ref[...],
        [q[...][:, :DIM] for q in (q0_ref, q1_ref, q2_ref)])
    zhat_ref[...] = z_hat


PADW = 128


def _make_sc_gather(B):
    info = plsc.get_sparse_core_info()
    NC, NS = info.num_cores, info.num_subcores
    NW = NC * NS
    b_per_w = B // NW
    ch = 512
    mesh = plsc.VectorSubcoreMesh(core_axis_name="c", subcore_axis_name="s")

    @functools.partial(
        pl.kernel, mesh=mesh,
        out_type=jax.ShapeDtypeStruct((B, PADW), jnp.float32),
        scratch_types=[
            pltpu.VMEM((b_per_w,), jnp.int32),
            pltpu.VMEM((ch, PADW), jnp.float32),
            pltpu.SemaphoreType.DMA,
        ],
    )
    def gather_k(table_hbm, idx_hbm, out_hbm, idx_v, rows_v, sem):
        wid = lax.axis_index("s") * NC + lax.axis_index("c")
        base = wid * b_per_w
        pltpu.sync_copy(idx_hbm.at[pl.ds(base, b_per_w)], idx_v)
        for c in range(b_per_w // ch):
            pltpu.async_copy(
                table_hbm.at[idx_v.at[pl.ds(c * ch, ch)]], rows_v, sem
            ).wait()
            pltpu.sync_copy(rows_v, out_hbm.at[pl.ds(base + c * ch, ch)])

    return gather_k


def _tc_level_call(nprev, zh, zqs, wt, ww, iotaf):
    Hl = zh.shape[0]
    tok_spec = pl.BlockSpec((BT, DIM), lambda i: (i, 0))
    zq_spec = pl.BlockSpec((BT, PADW), lambda i: (i, 0))
    wt_spec = pl.BlockSpec((DIM, K), lambda i: (0, 0))
    ww_spec = pl.BlockSpec((1, K), lambda i: (0, 0))
    idx_spec = pl.BlockSpec((BT, 1), lambda i: (i, 0))
    return pl.pallas_call(
        functools.partial(_level_block, nprev),
        grid=(Hl // BT,),
        in_specs=[tok_spec] + [zq_spec] * nprev
                 + [wt_spec, ww_spec, ww_spec],
        out_specs=idx_spec,
        out_shape=jax.ShapeDtypeStruct((Hl, 1), jnp.int32),
        compiler_params=pltpu.CompilerParams(
            dimension_semantics=("arbitrary",)),
    )(zh, *zqs, wt, ww, iotaf)


def _tc_combine_call(zh, q0, q1, q2):
    Hl = zh.shape[0]
    ctok = pl.BlockSpec((BTC, DIM), lambda i: (i, 0))
    cq = pl.BlockSpec((BTC, PADW), lambda i: (i, 0))
    return pl.pallas_call(
        _combine_block,
        grid=(Hl // BTC,),
        in_specs=[ctok, cq, cq, cq],
        out_specs=ctok,
        out_shape=jax.ShapeDtypeStruct((Hl, DIM), jnp.float32),
    )(zh, q0, q1, q2)


def kernel(z, codebook0, codebook1, codebook2):
    B = z.shape[0]
    cbs = (codebook0, codebook1, codebook2)
    cb_spec = pl.BlockSpec((K, DIM), lambda: (0, 0))
    ww_col = pl.pallas_call(
        _norms_block,
        in_specs=[cb_spec, cb_spec, cb_spec],
        out_specs=[pl.BlockSpec((K, 1), lambda: (0, 0))] * 3,
        out_shape=[jax.ShapeDtypeStruct((K, 1), jnp.float32)] * 3,
    )(*cbs)
    wws = [w.reshape(1, K) for w in ww_col]
    wts = [2.0 * cb.T for cb in cbs]
    iotaf = jnp.arange(K, dtype=jnp.float32).reshape(1, K)
    padded = [jnp.pad(cb, ((0, 0), (0, PADW - DIM))) for cb in cbs]

    H = B // 2
    sc_gather = _make_sc_gather(H)
    zhalf = [z[:H], z[H:]]

    idx = [[None, None, None] for _ in range(2)]
    zq = [[None, None, None] for _ in range(2)]
    for lvl in range(3):
        for h in range(2):
            idx[h][lvl] = _tc_level_call(lvl, zhalf[h], zq[h][:lvl],
                                         wts[lvl], wws[lvl], iotaf)
            zq[h][lvl] = sc_gather(padded[lvl], idx[h][lvl].reshape(H))

    zh_half = [_tc_combine_call(zhalf[h], *zq[h]) for h in range(2)]
    z_hat = jnp.concatenate(zh_half, axis=0)
    indices = jnp.concatenate(
        [jnp.concatenate(idx[h], axis=1) for h in range(2)], axis=0)
    return z_hat, indices


# speedup vs baseline: 2.2995x; 1.0210x over previous
import functools

import jax
import jax.numpy as jnp
from jax import lax
from jax.experimental import pallas as pl
from jax.experimental.pallas import tpu as pltpu
from jax.experimental.pallas import tpu_sc as plsc

K = 8192
DIM = 32
BT = 1024
BTC = 2048


def _norms_block(cb0_ref, cb1_ref, cb2_ref, w0_ref, w1_ref, w2_ref):
    for cb_ref, w_ref in ((cb0_ref, w0_ref), (cb1_ref, w1_ref),
                          (cb2_ref, w2_ref)):
        W = cb_ref[...]
        w_ref[...] = jnp.sum(W * W, axis=1, keepdims=True)


def _residual_chain(z, zqs):
    r = z
    z_hat = jnp.zeros_like(z)
    for zq in zqs:
        quant_st = r + (zq - r)
        z_hat = z_hat + quant_st
        r = r - quant_st
    return r, z_hat


def _level_block(nprev, *refs):
    z_ref = refs[0]
    zq_refs = refs[1:1 + nprev]
    wt_ref, ww_ref, iota_ref, i_ref = refs[1 + nprev:]
    r, _ = _residual_chain(z_ref[...],
                           [q[...][:, :DIM] for q in zq_refs])
    Wt2 = wt_ref[...]
    ww = ww_ref[...]
    iotaf = iota_ref[...]
    rr = jnp.sum(r * r, axis=1, keepdims=True)
    mm2 = jax.lax.dot_general(
        r, Wt2, (((1,), (0,)), ((), ())),
        preferred_element_type=jnp.float32)
    d2 = rr - mm2 + ww
    m = jnp.min(d2, axis=1, keepdims=True)
    idxf = jnp.min(jnp.where(d2 == m, iotaf, jnp.float32(K)),
                   axis=1, keepdims=True)
    i_ref[...] = idxf.astype(jnp.int32)


def _combine_block(z_ref, q0_ref, q1_ref, q2_ref, zhat_ref):
    _, z_hat = _residual_chain(
        z_ref[...],
        [q[...][:, :DIM] for q in (q0_ref, q1_ref, q2_ref)])
    zhat_ref[...] = z_hat


PADW = 128


def _make_sc_gather(B):
    info = plsc.get_sparse_core_info()
    NC, NS = info.num_cores, info.num_subcores
    NW = NC * NS
    b_per_w = B // NW
    ch = 512
    mesh = plsc.VectorSubcoreMesh(core_axis_name="c", subcore_axis_name="s")

    @functools.partial(
        pl.kernel, mesh=mesh,
        out_type=jax.ShapeDtypeStruct((B, PADW), jnp.float32),
        scratch_types=[
            pltpu.VMEM((b_per_w,), jnp.int32),
            pltpu.VMEM((ch, PADW), jnp.float32),
            pltpu.SemaphoreType.DMA,
        ],
    )
    def gather_k(table_hbm, idx_hbm, out_hbm, idx_v, rows_v, sem):
        wid = lax.axis_index("s") * NC + lax.axis_index("c")
        base = wid * b_per_w
        pltpu.sync_copy(idx_hbm.at[pl.ds(base, b_per_w)], idx_v)
        for c in range(b_per_w // ch):
            pltpu.async_copy(
                table_hbm.at[idx_v.at[pl.ds(c * ch, ch)]], rows_v, sem
            ).wait()
            pltpu.sync_copy(rows_v, out_hbm.at[pl.ds(base + c * ch, ch)])

    return gather_k


def _tc_level_call(nprev, zh, zqs, wt, ww, iotaf):
    Hl = zh.shape[0]
    tok_spec = pl.BlockSpec((BT, DIM), lambda i: (i, 0))
    zq_spec = pl.BlockSpec((BT, PADW), lambda i: (i, 0))
    wt_spec = pl.BlockSpec((DIM, K), lambda i: (0, 0))
    ww_spec = pl.BlockSpec((1, K), lambda i: (0, 0))
    idx_spec = pl.BlockSpec((BT, 1), lambda i: (i, 0))
    return pl.pallas_call(
        functools.partial(_level_block, nprev),
        grid=(Hl // BT,),
        in_specs=[tok_spec] + [zq_spec] * nprev
                 + [wt_spec, ww_spec, ww_spec],
        out_specs=idx_spec,
        out_shape=jax.ShapeDtypeStruct((Hl, 1), jnp.int32),
        compiler_params=pltpu.CompilerParams(
            dimension_semantics=("arbitrary",)),
    )(zh, *zqs, wt, ww, iotaf)


def _tc_combine_call(zh, q0, q1, q2):
    Hl = zh.shape[0]
    ctok = pl.BlockSpec((BTC, DIM), lambda i: (i, 0))
    cq = pl.BlockSpec((BTC, PADW), lambda i: (i, 0))
    return pl.pallas_call(
        _combine_block,
        grid=(Hl // BTC,),
        in_specs=[ctok, cq, cq, cq],
        out_specs=ctok,
        out_shape=jax.ShapeDtypeStruct((Hl, DIM), jnp.float32),
    )(zh, q0, q1, q2)


def kernel(z, codebook0, codebook1, codebook2):
    B = z.shape[0]
    cbs = (codebook0, codebook1, codebook2)
    cb_spec = pl.BlockSpec((K, DIM), lambda: (0, 0))
    ww_col = pl.pallas_call(
        _norms_block,
        in_specs=[cb_spec, cb_spec, cb_spec],
        out_specs=[pl.BlockSpec((K, 1), lambda: (0, 0))] * 3,
        out_shape=[jax.ShapeDtypeStruct((K, 1), jnp.float32)] * 3,
    )(*cbs)
    wws = [w.reshape(1, K) for w in ww_col]
    wts = [2.0 * cb.T for cb in cbs]
    iotaf = jnp.arange(K, dtype=jnp.float32).reshape(1, K)
    padded = [jnp.pad(cb, ((0, 0), (0, PADW - DIM))) for cb in cbs]

    H = B // 2
    sc_gather = _make_sc_gather(H)
    zhalf = [z[:H], z[H:]]

    idx = [[None, None, None] for _ in range(2)]
    zq = [[None, None, None] for _ in range(2)]
    for lvl in range(3):
        for h in range(2):
            idx[h][lvl] = _tc_level_call(lvl, zhalf[h], zq[h][:lvl],
                                         wts[lvl], wws[lvl], iotaf)
            zq[h][lvl] = sc_gather(padded[lvl], idx[h][lvl].reshape(H))

    zh_half = [_tc_combine_call(zhalf[h], *zq[h]) for h in range(2)]
    z_hat = jnp.concatenate(zh_half, axis=0)
    indices = jnp.concatenate(
        [jnp.concatenate(idx[h], axis=1) for h in range(2)], axis=0)
    return z_hat, indices
